# baseline (speedup 1.0000x reference)
import functools

import jax
import jax.numpy as jnp
from jax.experimental import pallas as pl
from jax.experimental.pallas import tpu as pltpu

_TILE_N = 2048


def _kmeans_tile(x_ref, c_ref, distT_ref, assign_ref, inertia_ref,
                 c2_ref, csq_ref, iota_ref):
    tn = x_ref.shape[0]
    k = c_ref.shape[0]

    @pl.when(pl.program_id(0) == 0)
    def _prep():
        c = c_ref[...]
        c2_ref[...] = c * -2.0
        csq_ref[...] = jnp.sum(c * c, axis=1, keepdims=True)
        iota_ref[...] = jax.lax.broadcasted_iota(
            jnp.int32, (k, tn), 0).astype(jnp.float32)

    x = x_ref[...]
    xT = x.T
    cross2 = jax.lax.dot_general(
        c2_ref[...], xT, (((1,), (0,)), ((), ())),
        preferred_element_type=jnp.float32)
    x_sq = jnp.sum(xT * xT, axis=0, keepdims=True)
    d2 = jnp.maximum((csq_ref[...] + x_sq) + cross2, 0.0)
    dist = jnp.sqrt(d2)
    distT_ref[...] = dist
    mn = jnp.min(d2, axis=0, keepdims=True)
    md = jnp.sqrt(mn)
    am = jnp.min(jnp.where(dist == md, iota_ref[...], float(k)), axis=0)
    assign_ref[...] = am.astype(jnp.int32)[None, None, :]
    inertia_ref[...] = (md * md)[None]


@functools.partial(jax.jit, static_argnames=())
def kernel(data, centroids):
    n, f = data.shape
    k = centroids.shape[0]
    g = n // _TILE_N
    distT, assign, inertia = pl.pallas_call(
        _kmeans_tile,
        grid=(g,),
        in_specs=[
            pl.BlockSpec((_TILE_N, f), lambda i: (i, 0)),
            pl.BlockSpec((k, f), lambda i: (0, 0)),
        ],
        out_specs=[
            pl.BlockSpec((k, _TILE_N), lambda i: (0, i)),
            pl.BlockSpec((1, 1, _TILE_N), lambda i: (i, 0, 0)),
            pl.BlockSpec((1, 1, _TILE_N), lambda i: (i, 0, 0)),
        ],
        out_shape=[
            jax.ShapeDtypeStruct((k, n), jnp.float32),
            jax.ShapeDtypeStruct((g, 1, _TILE_N), jnp.int32),
            jax.ShapeDtypeStruct((g, 1, _TILE_N), jnp.float32),
        ],
        compiler_params=pltpu.CompilerParams(
            vmem_limit_bytes=100 * 1024 * 1024,
        ),
        scratch_shapes=[
            pltpu.VMEM((k, f), jnp.float32),
            pltpu.VMEM((k, 1), jnp.float32),
            pltpu.VMEM((k, _TILE_N), jnp.float32),
        ],
    )(data, centroids)
    return distT.T, assign.reshape(n), inertia.reshape(n)


# speedup vs baseline: 1.0852x; 1.0852x over previous
import functools

import jax
import jax.numpy as jnp
from jax.experimental import pallas as pl
from jax.experimental.pallas import tpu as pltpu

_TILE_N = 2048


_CHUNK = 8


def _kmeans_tile(x_ref, c_ref, distT_ref, assign_ref, inertia_ref,
                 c2_ref, csq_ref):
    tn = x_ref.shape[0]
    k = c_ref.shape[0]

    @pl.when(pl.program_id(0) == 0)
    def _prep():
        c = c_ref[...]
        c2_ref[...] = c * -2.0
        csq_ref[...] = jnp.sum(c * c, axis=1, keepdims=True)

    x = x_ref[...]
    xT = x.T
    cross2 = jax.lax.dot_general(
        c2_ref[...], xT, (((1,), (0,)), ((), ())),
        preferred_element_type=jnp.float32)
    x_sq = jnp.sum(xT * xT, axis=0, keepdims=True)
    d2 = jnp.maximum((csq_ref[...] + x_sq) + cross2, 0.0)
    dist = jnp.sqrt(d2)
    distT_ref[...] = dist
    best = dist[0:_CHUNK, :]
    idx = jnp.zeros((_CHUNK, tn), jnp.float32)
    for r in range(1, k // _CHUNK):
        rows = dist[r * _CHUNK:(r + 1) * _CHUNK, :]
        cond = rows < best
        best = jnp.where(cond, rows, best)
        idx = jnp.where(cond, jnp.float32(r), idx)
    md = jnp.min(best, axis=0, keepdims=True)
    sub = jax.lax.broadcasted_iota(
        jnp.int32, (_CHUNK, tn), 0).astype(jnp.float32)
    cand = jnp.where(best == md, idx * float(_CHUNK) + sub, float(k))
    am = jnp.min(cand, axis=0)
    assign_ref[...] = am.astype(jnp.int32)[None, None, :]
    inertia_ref[...] = (md * md)[None]


@functools.partial(jax.jit, static_argnames=())
def kernel(data, centroids):
    n, f = data.shape
    k = centroids.shape[0]
    g = n // _TILE_N
    distT, assign, inertia = pl.pallas_call(
        _kmeans_tile,
        grid=(g,),
        in_specs=[
            pl.BlockSpec((_TILE_N, f), lambda i: (i, 0)),
            pl.BlockSpec((k, f), lambda i: (0, 0)),
        ],
        out_specs=[
            pl.BlockSpec((k, _TILE_N), lambda i: (0, i)),
            pl.BlockSpec((1, 1, _TILE_N), lambda i: (i, 0, 0)),
            pl.BlockSpec((1, 1, _TILE_N), lambda i: (i, 0, 0)),
        ],
        out_shape=[
            jax.ShapeDtypeStruct((k, n), jnp.float32),
            jax.ShapeDtypeStruct((g, 1, _TILE_N), jnp.int32),
            jax.ShapeDtypeStruct((g, 1, _TILE_N), jnp.float32),
        ],
        compiler_params=pltpu.CompilerParams(
            vmem_limit_bytes=100 * 1024 * 1024,
        ),
        scratch_shapes=[
            pltpu.VMEM((k, f), jnp.float32),
            pltpu.VMEM((k, 1), jnp.float32),
        ],
    )(data, centroids)
    return distT.T, assign.reshape(n), inertia.reshape(n)


# speedup vs baseline: 1.3343x; 1.2296x over previous
import functools

import jax
import jax.numpy as jnp
from jax.experimental import pallas as pl
from jax.experimental.pallas import tpu as pltpu

_TILE_N = 2048


_CHUNK = 8


def _kmeans_tile(x_ref, c_ref, distT_ref, assign_ref, inertia_ref,
                 c2_ref, csq_ref):
    tn = x_ref.shape[0]
    k = c_ref.shape[0]

    @pl.when(pl.program_id(0) == 0)
    def _prep():
        c = c_ref[...]
        c2_ref[...] = c * -2.0
        csq_ref[...] = jnp.sum(c * c, axis=1, keepdims=True)

    x = x_ref[...]
    xT = x.T
    cross2 = jax.lax.dot_general(
        c2_ref[...], xT, (((1,), (0,)), ((), ())),
        preferred_element_type=jnp.float32)
    x_sq = jnp.sum(xT * xT, axis=0, keepdims=True)
    d2 = jnp.maximum((csq_ref[...] + x_sq) + cross2, 1.1754944e-38)
    dist = d2 * jax.lax.rsqrt(d2)
    distT_ref[...] = dist
    best = dist[0:_CHUNK, :]
    idx = jnp.zeros((_CHUNK, tn), jnp.float32)
    for r in range(1, k // _CHUNK):
        rows = dist[r * _CHUNK:(r + 1) * _CHUNK, :]
        cond = rows < best
        best = jnp.where(cond, rows, best)
        idx = jnp.where(cond, jnp.float32(r), idx)
    md = jnp.min(best, axis=0, keepdims=True)
    sub = jax.lax.broadcasted_iota(
        jnp.int32, (_CHUNK, tn), 0).astype(jnp.float32)
    cand = jnp.where(best == md, idx * float(_CHUNK) + sub, float(k))
    am = jnp.min(cand, axis=0)
    assign_ref[...] = am.astype(jnp.int32)[None, None, :]
    inertia_ref[...] = (md * md)[None]


@functools.partial(jax.jit, static_argnames=())
def kernel(data, centroids):
    n, f = data.shape
    k = centroids.shape[0]
    g = n // _TILE_N
    distT, assign, inertia = pl.pallas_call(
        _kmeans_tile,
        grid=(g,),
        in_specs=[
            pl.BlockSpec((_TILE_N, f), lambda i: (i, 0)),
            pl.BlockSpec((k, f), lambda i: (0, 0)),
        ],
        out_specs=[
            pl.BlockSpec((k, _TILE_N), lambda i: (0, i)),
            pl.BlockSpec((1, 1, _TILE_N), lambda i: (i, 0, 0)),
            pl.BlockSpec((1, 1, _TILE_N), lambda i: (i, 0, 0)),
        ],
        out_shape=[
            jax.ShapeDtypeStruct((k, n), jnp.float32),
            jax.ShapeDtypeStruct((g, 1, _TILE_N), jnp.int32),
            jax.ShapeDtypeStruct((g, 1, _TILE_N), jnp.float32),
        ],
        compiler_params=pltpu.CompilerParams(
            vmem_limit_bytes=100 * 1024 * 1024,
        ),
        scratch_shapes=[
            pltpu.VMEM((k, f), jnp.float32),
            pltpu.VMEM((k, 1), jnp.float32),
        ],
    )(data, centroids)
    return distT.T, assign.reshape(n), inertia.reshape(n)
